# baseline (speedup 1.0000x reference)
import functools

import jax
import jax.numpy as jnp
from jax import lax
from jax.experimental import pallas as pl
from jax.experimental.pallas import tpu as pltpu
from jax.experimental.pallas import tpu_sc as plsc

N = 10000
NP = 10240
E = 320000
D = 128
NC = 2
NS = 16
CHUNK = 128
EPT = 10240
EP = EPT * NC * NS
NCHUNK = EPT // CHUNK
RPT = NP // NS

_mesh = plsc.VectorSubcoreMesh(core_axis_name="c", subcore_axis_name="s")


@functools.partial(
    pl.kernel,
    out_type=jax.ShapeDtypeStruct((NC, NP), jnp.float32),
    mesh=_mesh,
    scratch_types=[
        pltpu.VMEM((CHUNK,), jnp.int32),
        pltpu.VMEM((CHUNK,), jnp.float32),
        pltpu.VMEM((RPT,), jnp.float32),
        pltpu.VMEM_SHARED((NP,), jnp.float32),
    ],
)
def _deg_kernel(dst_hbm, deg_hbm, idx_v, ones_v, zer_v, deg_sp):
    c = lax.axis_index("c")
    s = lax.axis_index("s")
    wid = c * NS + s
    for i in range(CHUNK // 16):
        ones_v[pl.ds(i * 16, 16)] = jnp.ones((16,), jnp.float32)
    for i in range(RPT // 16):
        zer_v[pl.ds(i * 16, 16)] = jnp.zeros((16,), jnp.float32)
    pltpu.sync_copy(zer_v, deg_sp.at[pl.ds(s * RPT, RPT)])
    plsc.subcore_barrier()

    base = wid * EPT

    def step(j, carry):
        off = pl.multiple_of(base + j * CHUNK, 8)
        pltpu.sync_copy(dst_hbm.at[pl.ds(off, CHUNK)], idx_v)
        pltpu.sync_copy(ones_v, deg_sp.at[idx_v], add=True)
        return carry

    lax.fori_loop(0, NCHUNK, step, 0)
    plsc.subcore_barrier()
    pltpu.sync_copy(deg_sp.at[pl.ds(s * RPT, RPT)],
                    deg_hbm.at[c, pl.ds(s * RPT, RPT)])


@functools.partial(
    pl.kernel,
    out_type=jax.ShapeDtypeStruct((NC, NP, D), jnp.float32),
    mesh=_mesh,
    scratch_types=[
        pltpu.VMEM((CHUNK,), jnp.int32),
        pltpu.VMEM((CHUNK,), jnp.int32),
        pltpu.VMEM((CHUNK, D), jnp.float32),
        pltpu.VMEM_SHARED((NP, D), jnp.float32),
        pltpu.SemaphoreType.DMA,
    ],
)
def _msg_kernel(src_hbm, dst_hbm, xlin_hbm, z_hbm, sidx, didx, rows, z_sp, sem):
    c = lax.axis_index("c")
    s = lax.axis_index("s")
    wid = c * NS + s

    def zrow(r, carry):
        for i in range(D // 16):
            rows[r, pl.ds(i * 16, 16)] = jnp.zeros((16,), jnp.float32)
        return carry

    lax.fori_loop(0, CHUNK, zrow, 0)
    for k in range(RPT // CHUNK):
        pltpu.sync_copy(rows, z_sp.at[pl.ds(s * RPT + k * CHUNK, CHUNK)])
    plsc.subcore_barrier()

    base = wid * EPT

    def step(j, carry):
        off = pl.multiple_of(base + j * CHUNK, 8)
        pltpu.sync_copy(src_hbm.at[pl.ds(off, CHUNK)], sidx)
        pltpu.sync_copy(dst_hbm.at[pl.ds(off, CHUNK)], didx)
        pltpu.async_copy(xlin_hbm.at[sidx], rows, sem).wait()
        pltpu.sync_copy(rows, z_sp.at[didx], add=True)
        return carry

    lax.fori_loop(0, NCHUNK, step, 0)
    plsc.subcore_barrier()
    pltpu.sync_copy(z_sp.at[pl.ds(s * RPT, RPT)],
                    z_hbm.at[c, pl.ds(s * RPT, RPT)])


def _lin_body(x_ref, w_ref, degt_ref, o_ref):
    deg = degt_ref[:, 0:1] + degt_ref[:, 1:2] + 1.0
    dinv = lax.rsqrt(deg)
    o_ref[:] = jnp.dot(x_ref[:] * dinv, w_ref[:],
                       preferred_element_type=jnp.float32)


def _out_body(z_ref, xlin_ref, degt_ref, b_ref, o_ref):
    deg = degt_ref[:, 0:1] + degt_ref[:, 1:2] + 1.0
    dinv = lax.rsqrt(deg)
    zsum = z_ref[0] + z_ref[1] + xlin_ref[:]
    o_ref[:] = jnp.maximum(zsum * dinv + b_ref[:], 0.0)


def kernel(x, edge_index, W, b):
    ei = edge_index.astype(jnp.int32)
    src = jnp.concatenate([ei[0], jnp.zeros((EP - E,), jnp.int32)])
    dst = jnp.concatenate([ei[1], jnp.full((EP - E,), N, jnp.int32)])
    x_p = jnp.pad(x, ((0, NP - N), (0, 0)))

    deg2 = _deg_kernel(dst)
    degt = deg2.T

    xlin = pl.pallas_call(
        _lin_body,
        out_shape=jax.ShapeDtypeStruct((NP, D), jnp.float32),
    )(x_p, W, degt)

    z2 = _msg_kernel(src, dst, xlin)

    out = pl.pallas_call(
        _out_body,
        out_shape=jax.ShapeDtypeStruct((NP, D), jnp.float32),
    )(z2, xlin, degt, b.reshape(1, D))
    return out[:N]


# speedup vs baseline: 1.5005x; 1.5005x over previous
import functools

import jax
import jax.numpy as jnp
from jax import lax
from jax.experimental import pallas as pl
from jax.experimental.pallas import tpu as pltpu
from jax.experimental.pallas import tpu_sc as plsc

N = 10000
NP = 10240
E = 320000
D = 128
NC = 2
NS = 16
CHUNK = 128
EPT = 10112
EP = EPT * NC * NS
NCHUNK = EPT // CHUNK
RPT = NP // NS

_mesh = plsc.VectorSubcoreMesh(core_axis_name="c", subcore_axis_name="s")


@functools.partial(
    pl.kernel,
    out_type=jax.ShapeDtypeStruct((NC, NP), jnp.float32),
    mesh=_mesh,
    scratch_types=[
        pltpu.VMEM((NCHUNK, CHUNK), jnp.int32),
        pltpu.VMEM((CHUNK,), jnp.float32),
        pltpu.VMEM((RPT,), jnp.float32),
        pltpu.VMEM_SHARED((NP,), jnp.float32),
        pltpu.SemaphoreType.DMA,
    ],
)
def _deg_kernel(dst_hbm, deg_hbm, didx, ones_v, zer_v, deg_sp, sem):
    c = lax.axis_index("c")
    s = lax.axis_index("s")
    wid = c * NS + s
    for i in range(CHUNK // 16):
        ones_v[pl.ds(i * 16, 16)] = jnp.ones((16,), jnp.float32)
    for i in range(RPT // 16):
        zer_v[pl.ds(i * 16, 16)] = jnp.zeros((16,), jnp.float32)
    pltpu.sync_copy(zer_v, deg_sp.at[pl.ds(s * RPT, RPT)])
    pltpu.sync_copy(dst_hbm.at[wid], didx)
    plsc.subcore_barrier()

    def fire(j, carry):
        pltpu.async_copy(ones_v, deg_sp.at[didx.at[j]], sem, add=True)
        return carry

    lax.fori_loop(0, NCHUNK, fire, 0)

    def drain(j, carry):
        pltpu.make_async_copy(ones_v, deg_sp.at[didx.at[j]], sem).wait()
        return carry

    lax.fori_loop(0, NCHUNK, drain, 0)
    plsc.subcore_barrier()
    pltpu.sync_copy(deg_sp.at[pl.ds(s * RPT, RPT)],
                    deg_hbm.at[c, pl.ds(s * RPT, RPT)])


@functools.partial(
    pl.kernel,
    out_type=jax.ShapeDtypeStruct((NC, NP, D), jnp.float32),
    mesh=_mesh,
    scratch_types=[
        pltpu.VMEM((CHUNK,), jnp.int32),
        pltpu.VMEM((CHUNK,), jnp.int32),
        pltpu.VMEM((CHUNK, D), jnp.float32),
        pltpu.VMEM_SHARED((NP, D), jnp.float32),
        pltpu.SemaphoreType.DMA,
    ],
)
def _msg_kernel(src_hbm, dst_hbm, xlin_hbm, z_hbm, sidx, didx, rows, z_sp, sem):
    c = lax.axis_index("c")
    s = lax.axis_index("s")
    wid = c * NS + s

    def zrow(r, carry):
        for i in range(D // 16):
            rows[r, pl.ds(i * 16, 16)] = jnp.zeros((16,), jnp.float32)
        return carry

    lax.fori_loop(0, CHUNK, zrow, 0)
    for k in range(RPT // CHUNK):
        pltpu.sync_copy(rows, z_sp.at[pl.ds(s * RPT + k * CHUNK, CHUNK)])
    plsc.subcore_barrier()

    base = wid * EPT

    def step(j, carry):
        off = pl.multiple_of(base + j * CHUNK, 8)
        pltpu.sync_copy(src_hbm.at[pl.ds(off, CHUNK)], sidx)
        pltpu.sync_copy(dst_hbm.at[pl.ds(off, CHUNK)], didx)
        pltpu.async_copy(xlin_hbm.at[sidx], rows, sem).wait()
        pltpu.sync_copy(rows, z_sp.at[didx], add=True)
        return carry

    lax.fori_loop(0, NCHUNK, step, 0)
    plsc.subcore_barrier()
    pltpu.sync_copy(z_sp.at[pl.ds(s * RPT, RPT)],
                    z_hbm.at[c, pl.ds(s * RPT, RPT)])


def _lin_body(x_ref, w_ref, degt_ref, o_ref):
    deg = degt_ref[:, 0:1] + degt_ref[:, 1:2] + 1.0
    dinv = lax.rsqrt(deg)
    o_ref[:] = jnp.dot(x_ref[:] * dinv, w_ref[:],
                       preferred_element_type=jnp.float32)


def _out_body(z_ref, xlin_ref, degt_ref, b_ref, o_ref):
    deg = degt_ref[:, 0:1] + degt_ref[:, 1:2] + 1.0
    dinv = lax.rsqrt(deg)
    zsum = z_ref[0] + z_ref[1] + xlin_ref[:]
    o_ref[:] = jnp.maximum(zsum * dinv + b_ref[:], 0.0)


def kernel(x, edge_index, W, b):
    ei = edge_index.astype(jnp.int32)
    nw = NC * NS
    src = jnp.concatenate([ei[0], jnp.zeros((EP - E,), jnp.int32)])
    dst = jnp.concatenate([ei[1], jnp.full((EP - E,), N, jnp.int32)])
    src3 = src.reshape(NCHUNK, nw, CHUNK).swapaxes(0, 1)
    dst3 = dst.reshape(NCHUNK, nw, CHUNK).swapaxes(0, 1)
    src = src3.reshape(EP)
    dst = dst3.reshape(EP)
    x_p = jnp.pad(x, ((0, NP - N), (0, 0)))

    deg2 = _deg_kernel(dst3)
    degt = deg2.T

    xlin = pl.pallas_call(
        _lin_body,
        out_shape=jax.ShapeDtypeStruct((NP, D), jnp.float32),
    )(x_p, W, degt)

    z2 = _msg_kernel(src, dst, xlin)

    out = pl.pallas_call(
        _out_body,
        out_shape=jax.ShapeDtypeStruct((NP, D), jnp.float32),
    )(z2, xlin, degt, b.reshape(1, D))
    return out[:N]


# speedup vs baseline: 2.0427x; 1.3614x over previous
import functools

import jax
import jax.numpy as jnp
from jax import lax
from jax.experimental import pallas as pl
from jax.experimental.pallas import tpu as pltpu
from jax.experimental.pallas import tpu_sc as plsc

N = 10000
NP = 10240
E = 320000
D = 128
NC = 2
NS = 16
CHUNK = 128
EPT = 10112
EP = EPT * NC * NS
NCHUNK = EPT // CHUNK
RPT = NP // NS

_mesh = plsc.VectorSubcoreMesh(core_axis_name="c", subcore_axis_name="s")


@functools.partial(
    pl.kernel,
    out_type=jax.ShapeDtypeStruct((NC, NP), jnp.float32),
    mesh=_mesh,
    scratch_types=[
        pltpu.VMEM((NCHUNK, CHUNK), jnp.int32),
        pltpu.VMEM((CHUNK,), jnp.float32),
        pltpu.VMEM((RPT,), jnp.float32),
        pltpu.VMEM_SHARED((NP,), jnp.float32),
        pltpu.SemaphoreType.DMA,
    ],
)
def _deg_kernel(dst_hbm, deg_hbm, didx, ones_v, zer_v, deg_sp, sem):
    c = lax.axis_index("c")
    s = lax.axis_index("s")
    wid = c * NS + s
    for i in range(CHUNK // 16):
        ones_v[pl.ds(i * 16, 16)] = jnp.ones((16,), jnp.float32)
    for i in range(RPT // 16):
        zer_v[pl.ds(i * 16, 16)] = jnp.zeros((16,), jnp.float32)
    pltpu.sync_copy(zer_v, deg_sp.at[pl.ds(s * RPT, RPT)])
    pltpu.sync_copy(dst_hbm.at[wid], didx)
    plsc.subcore_barrier()

    def fire(j, carry):
        pltpu.async_copy(ones_v, deg_sp.at[didx.at[j]], sem, add=True)
        return carry

    lax.fori_loop(0, NCHUNK, fire, 0)

    def drain(j, carry):
        pltpu.make_async_copy(ones_v, deg_sp.at[didx.at[j]], sem).wait()
        return carry

    lax.fori_loop(0, NCHUNK, drain, 0)
    plsc.subcore_barrier()
    pltpu.sync_copy(deg_sp.at[pl.ds(s * RPT, RPT)],
                    deg_hbm.at[c, pl.ds(s * RPT, RPT)])


@functools.partial(
    pl.kernel,
    out_type=jax.ShapeDtypeStruct((NC, NP, D), jnp.float32),
    mesh=_mesh,
    scratch_types=[
        pltpu.VMEM((NCHUNK, CHUNK), jnp.int32),
        pltpu.VMEM((CHUNK,), jnp.int32),
        [pltpu.VMEM((CHUNK, D), jnp.float32)] * 2,
        [pltpu.SemaphoreType.DMA] * 2,
        pltpu.VMEM_SHARED((NP, D), jnp.float32),
    ],
)
def _msg_kernel(src_hbm, dst_hbm, xlin_hbm, z_hbm, sidx, didx, rows,
                gsem, z_sp):
    c = lax.axis_index("c")
    s = lax.axis_index("s")
    wid = c * NS + s

    def zrow(r, carry):
        for i in range(D // 16):
            rows[0][r, pl.ds(i * 16, 16)] = jnp.zeros((16,), jnp.float32)
        return carry

    lax.fori_loop(0, CHUNK, zrow, 0)
    for k in range(RPT // CHUNK):
        pltpu.sync_copy(rows[0], z_sp.at[pl.ds(s * RPT + k * CHUNK, CHUNK)])
    pltpu.sync_copy(src_hbm.at[wid], sidx)
    plsc.subcore_barrier()

    base = wid * EPT
    for b in range(2):
        pltpu.async_copy(xlin_hbm.at[sidx.at[b]], rows[b], gsem[b])

    def body(j, b):
        pltpu.make_async_copy(xlin_hbm.at[sidx.at[j]], rows[b],
                              gsem[b]).wait()
        off = pl.multiple_of(base + j * CHUNK, 8)
        pltpu.sync_copy(dst_hbm.at[pl.ds(off, CHUNK)], didx)
        pltpu.sync_copy(rows[b], z_sp.at[didx], add=True)

        @pl.when(j + 2 < NCHUNK)
        def _():
            pltpu.async_copy(
                xlin_hbm.at[sidx.at[jnp.minimum(j + 2, NCHUNK - 1)]],
                rows[b], gsem[b])

    def stage(jo, carry):
        body(jo * 2, 0)
        body(jo * 2 + 1, 1)
        return carry

    lax.fori_loop(0, NCHUNK // 2, stage, 0)
    body(NCHUNK - 1, 0)
    plsc.subcore_barrier()
    pltpu.sync_copy(z_sp.at[pl.ds(s * RPT, RPT)],
                    z_hbm.at[c, pl.ds(s * RPT, RPT)])


def _lin_body(x_ref, w_ref, degt_ref, o_ref):
    deg = degt_ref[:, 0:1] + degt_ref[:, 1:2] + 1.0
    dinv = lax.rsqrt(deg)
    o_ref[:] = jnp.dot(x_ref[:] * dinv, w_ref[:],
                       preferred_element_type=jnp.float32)


def _out_body(z_ref, xlin_ref, degt_ref, b_ref, o_ref):
    deg = degt_ref[:, 0:1] + degt_ref[:, 1:2] + 1.0
    dinv = lax.rsqrt(deg)
    zsum = z_ref[0] + z_ref[1] + xlin_ref[:]
    o_ref[:] = jnp.maximum(zsum * dinv + b_ref[:], 0.0)


def kernel(x, edge_index, W, b):
    ei = edge_index.astype(jnp.int32)
    nw = NC * NS
    src = jnp.concatenate([ei[0], jnp.zeros((EP - E,), jnp.int32)])
    dst = jnp.concatenate([ei[1], jnp.full((EP - E,), N, jnp.int32)])
    src3 = src.reshape(NCHUNK, nw, CHUNK).swapaxes(0, 1)
    dst3 = dst.reshape(NCHUNK, nw, CHUNK).swapaxes(0, 1)
    src = src3.reshape(EP)
    dst = dst3.reshape(EP)
    x_p = jnp.pad(x, ((0, NP - N), (0, 0)))

    deg2 = _deg_kernel(dst3)
    degt = deg2.T

    xlin = pl.pallas_call(
        _lin_body,
        out_shape=jax.ShapeDtypeStruct((NP, D), jnp.float32),
    )(x_p, W, degt)

    z2 = _msg_kernel(src3, dst, xlin)

    out = pl.pallas_call(
        _out_body,
        out_shape=jax.ShapeDtypeStruct((NP, D), jnp.float32),
    )(z2, xlin, degt, b.reshape(1, D))
    return out[:N]


# speedup vs baseline: 2.1498x; 1.0524x over previous
import functools

import jax
import jax.numpy as jnp
from jax import lax
from jax.experimental import pallas as pl
from jax.experimental.pallas import tpu as pltpu
from jax.experimental.pallas import tpu_sc as plsc

N = 10000
NP = 10240
E = 320000
D = 128
NC = 2
NS = 16
CHUNK = 128
EPT = 10112
EP = EPT * NC * NS
NCHUNK = EPT // CHUNK
RPT = NP // NS

_mesh = plsc.VectorSubcoreMesh(core_axis_name="c", subcore_axis_name="s")


@functools.partial(
    pl.kernel,
    out_type=jax.ShapeDtypeStruct((NC, NP), jnp.float32),
    mesh=_mesh,
    scratch_types=[
        pltpu.VMEM((NCHUNK, CHUNK), jnp.int32),
        pltpu.VMEM((CHUNK,), jnp.float32),
        pltpu.VMEM((RPT,), jnp.float32),
        pltpu.VMEM_SHARED((NP,), jnp.float32),
        pltpu.SemaphoreType.DMA,
    ],
)
def _deg_kernel(dst_hbm, deg_hbm, didx, ones_v, zer_v, deg_sp, sem):
    c = lax.axis_index("c")
    s = lax.axis_index("s")
    wid = c * NS + s
    for i in range(CHUNK // 16):
        ones_v[pl.ds(i * 16, 16)] = jnp.ones((16,), jnp.float32)
    for i in range(RPT // 16):
        zer_v[pl.ds(i * 16, 16)] = jnp.zeros((16,), jnp.float32)
    pltpu.sync_copy(zer_v, deg_sp.at[pl.ds(s * RPT, RPT)])
    pltpu.sync_copy(dst_hbm.at[wid], didx)
    plsc.subcore_barrier()

    def fire(j, carry):
        pltpu.async_copy(ones_v, deg_sp.at[didx.at[j]], sem, add=True)
        return carry

    lax.fori_loop(0, NCHUNK, fire, 0)

    def drain(j, carry):
        pltpu.make_async_copy(ones_v, deg_sp.at[didx.at[j]], sem).wait()
        return carry

    lax.fori_loop(0, NCHUNK, drain, 0)
    plsc.subcore_barrier()
    pltpu.sync_copy(deg_sp.at[pl.ds(s * RPT, RPT)],
                    deg_hbm.at[c, pl.ds(s * RPT, RPT)])


@functools.partial(
    pl.kernel,
    out_type=jax.ShapeDtypeStruct((NC, NP, D), jnp.float32),
    mesh=_mesh,
    scratch_types=[
        pltpu.VMEM((NCHUNK, CHUNK), jnp.int32),
        [pltpu.VMEM((CHUNK,), jnp.int32)] * 2,
        [pltpu.VMEM((CHUNK, D), jnp.float32)] * 2,
        [pltpu.SemaphoreType.DMA] * 2,
        [pltpu.SemaphoreType.DMA] * 2,
        pltpu.VMEM_SHARED((NP, D), jnp.float32),
    ],
)
def _msg_kernel(src_hbm, dst_hbm, xlin_hbm, z_hbm, sidx, didx, rows,
                gsem, isem, z_sp):
    c = lax.axis_index("c")
    s = lax.axis_index("s")
    wid = c * NS + s

    def zrow(r, carry):
        for i in range(D // 16):
            rows[0][r, pl.ds(i * 16, 16)] = jnp.zeros((16,), jnp.float32)
        return carry

    lax.fori_loop(0, CHUNK, zrow, 0)
    for k in range(RPT // CHUNK):
        pltpu.sync_copy(rows[0], z_sp.at[pl.ds(s * RPT + k * CHUNK, CHUNK)])
    pltpu.sync_copy(src_hbm.at[wid], sidx)
    plsc.subcore_barrier()

    base = wid * EPT
    for b in range(2):
        pltpu.async_copy(xlin_hbm.at[sidx.at[b]], rows[b], gsem[b])
        off0 = pl.multiple_of(base + b * CHUNK, 8)
        pltpu.async_copy(dst_hbm.at[pl.ds(off0, CHUNK)], didx[b], isem[b])

    def body(j, b):
        off = pl.multiple_of(base + j * CHUNK, 8)
        pltpu.make_async_copy(dst_hbm.at[pl.ds(off, CHUNK)], didx[b],
                              isem[b]).wait()
        pltpu.make_async_copy(xlin_hbm.at[sidx.at[j]], rows[b],
                              gsem[b]).wait()
        pltpu.sync_copy(rows[b], z_sp.at[didx[b]], add=True)

        @pl.when(j + 2 < NCHUNK)
        def _():
            jn = jnp.minimum(j + 2, NCHUNK - 1)
            offn = pl.multiple_of(base, 8) + jn * CHUNK
            pltpu.async_copy(xlin_hbm.at[sidx.at[jn]], rows[b], gsem[b])
            pltpu.async_copy(dst_hbm.at[pl.ds(offn, CHUNK)], didx[b],
                             isem[b])

    def stage(jo, carry):
        body(jo * 2, 0)
        body(jo * 2 + 1, 1)
        return carry

    lax.fori_loop(0, NCHUNK // 2, stage, 0)
    body(NCHUNK - 1, 0)
    plsc.subcore_barrier()
    pltpu.sync_copy(z_sp.at[pl.ds(s * RPT, RPT)],
                    z_hbm.at[c, pl.ds(s * RPT, RPT)])


def _lin_body(x_ref, w_ref, degt_ref, o_ref):
    deg = degt_ref[:, 0:1] + degt_ref[:, 1:2] + 1.0
    dinv = lax.rsqrt(deg)
    o_ref[:] = jnp.dot(x_ref[:] * dinv, w_ref[:],
                       preferred_element_type=jnp.float32)


def _out_body(z_ref, xlin_ref, degt_ref, b_ref, o_ref):
    deg = degt_ref[:, 0:1] + degt_ref[:, 1:2] + 1.0
    dinv = lax.rsqrt(deg)
    zsum = z_ref[0] + z_ref[1] + xlin_ref[:]
    o_ref[:] = jnp.maximum(zsum * dinv + b_ref[:], 0.0)


def kernel(x, edge_index, W, b):
    ei = edge_index.astype(jnp.int32)
    nw = NC * NS
    src = jnp.concatenate([ei[0], jnp.zeros((EP - E,), jnp.int32)])
    dst = jnp.concatenate([ei[1], jnp.full((EP - E,), N, jnp.int32)])
    src3 = src.reshape(NCHUNK, nw, CHUNK).swapaxes(0, 1)
    dst3 = dst.reshape(NCHUNK, nw, CHUNK).swapaxes(0, 1)
    src = src3.reshape(EP)
    dst = dst3.reshape(EP)
    x_p = jnp.pad(x, ((0, NP - N), (0, 0)))

    deg2 = _deg_kernel(dst3)
    degt = deg2.T

    xlin = pl.pallas_call(
        _lin_body,
        out_shape=jax.ShapeDtypeStruct((NP, D), jnp.float32),
    )(x_p, W, degt)

    z2 = _msg_kernel(src3, dst, xlin)

    out = pl.pallas_call(
        _out_body,
        out_shape=jax.ShapeDtypeStruct((NP, D), jnp.float32),
    )(z2, xlin, degt, b.reshape(1, D))
    return out[:N]


# speedup vs baseline: 2.1773x; 1.0128x over previous
import functools

import jax
import jax.numpy as jnp
from jax import lax
from jax.experimental import pallas as pl
from jax.experimental.pallas import tpu as pltpu
from jax.experimental.pallas import tpu_sc as plsc

N = 10000
NP = 10240
E = 320000
D = 128
NC = 2
NS = 16
CHUNK = 128
EPT = 10112
EP = EPT * NC * NS
NCHUNK = EPT // CHUNK
RPT = NP // NS

_mesh = plsc.VectorSubcoreMesh(core_axis_name="c", subcore_axis_name="s")


@functools.partial(
    pl.kernel,
    out_type=jax.ShapeDtypeStruct((NC, NP), jnp.float32),
    mesh=_mesh,
    scratch_types=[
        pltpu.VMEM((NCHUNK, CHUNK), jnp.int32),
        pltpu.VMEM((CHUNK,), jnp.float32),
        pltpu.VMEM((RPT,), jnp.float32),
        pltpu.VMEM_SHARED((NP,), jnp.float32),
        pltpu.SemaphoreType.DMA,
    ],
)
def _deg_kernel(dst_hbm, deg_hbm, didx, ones_v, zer_v, deg_sp, sem):
    c = lax.axis_index("c")
    s = lax.axis_index("s")
    wid = c * NS + s
    for i in range(CHUNK // 16):
        ones_v[pl.ds(i * 16, 16)] = jnp.ones((16,), jnp.float32)
    for i in range(RPT // 16):
        zer_v[pl.ds(i * 16, 16)] = jnp.zeros((16,), jnp.float32)
    pltpu.sync_copy(zer_v, deg_sp.at[pl.ds(s * RPT, RPT)])
    pltpu.sync_copy(dst_hbm.at[wid], didx)
    plsc.subcore_barrier()

    def fire(j, carry):
        pltpu.async_copy(ones_v, deg_sp.at[didx.at[j]], sem, add=True)
        return carry

    lax.fori_loop(0, NCHUNK, fire, 0)

    def drain(j, carry):
        pltpu.make_async_copy(ones_v, deg_sp.at[didx.at[j]], sem).wait()
        return carry

    lax.fori_loop(0, NCHUNK, drain, 0)
    plsc.subcore_barrier()
    pltpu.sync_copy(deg_sp.at[pl.ds(s * RPT, RPT)],
                    deg_hbm.at[c, pl.ds(s * RPT, RPT)])


@functools.partial(
    pl.kernel,
    out_type=jax.ShapeDtypeStruct((NC, NP, D), jnp.float32),
    mesh=_mesh,
    scratch_types=[
        pltpu.VMEM((NCHUNK, CHUNK), jnp.int32),
        [pltpu.VMEM((CHUNK,), jnp.int32)] * 2,
        [pltpu.VMEM((CHUNK, D), jnp.float32)] * 2,
        [pltpu.SemaphoreType.DMA] * 2,
        [pltpu.SemaphoreType.DMA] * 2,
        pltpu.VMEM_SHARED((NP, D), jnp.float32),
    ],
)
def _msg_kernel(src_hbm, dst_hbm, xlin_hbm, z_hbm, sidx, didx, rows,
                gsem, isem, z_sp):
    c = lax.axis_index("c")
    s = lax.axis_index("s")
    wid = c * NS + s

    def zrow(r, carry):
        for i in range(D // 16):
            rows[0][r, pl.ds(i * 16, 16)] = jnp.zeros((16,), jnp.float32)
        return carry

    lax.fori_loop(0, CHUNK, zrow, 0)
    for k in range(RPT // CHUNK):
        pltpu.sync_copy(rows[0], z_sp.at[pl.ds(s * RPT + k * CHUNK, CHUNK)])
    pltpu.sync_copy(src_hbm.at[wid], sidx)
    plsc.subcore_barrier()

    base = wid * EPT
    for b in range(2):
        pltpu.async_copy(xlin_hbm.at[sidx.at[b]], rows[b], gsem[b])
        off0 = pl.multiple_of(base + b * CHUNK, 8)
        pltpu.async_copy(dst_hbm.at[pl.ds(off0, CHUNK)], didx[b], isem[b])

    def body(j, b):
        off = pl.multiple_of(base + j * CHUNK, 8)
        pltpu.make_async_copy(dst_hbm.at[pl.ds(off, CHUNK)], didx[b],
                              isem[b]).wait()
        pltpu.make_async_copy(xlin_hbm.at[sidx.at[j]], rows[b],
                              gsem[b]).wait()
        pltpu.sync_copy(rows[b], z_sp.at[didx[b]], add=True)

        @pl.when(j + 2 < NCHUNK)
        def _():
            jn = jnp.minimum(j + 2, NCHUNK - 1)
            offn = pl.multiple_of(base, 8) + jn * CHUNK
            pltpu.async_copy(xlin_hbm.at[sidx.at[jn]], rows[b], gsem[b])
            pltpu.async_copy(dst_hbm.at[pl.ds(offn, CHUNK)], didx[b],
                             isem[b])

    def stage(jo, carry):
        body(jo * 2, 0)
        body(jo * 2 + 1, 1)
        return carry

    lax.fori_loop(0, NCHUNK // 2, stage, 0)
    body(NCHUNK - 1, 0)
    plsc.subcore_barrier()
    pltpu.sync_copy(z_sp.at[pl.ds(s * RPT, RPT)],
                    z_hbm.at[c, pl.ds(s * RPT, RPT)])


def _lin_body(x_ref, w_ref, degt_ref, o_ref):
    deg = degt_ref[:N, 0:1] + degt_ref[:N, 1:2] + 1.0
    dinv = lax.rsqrt(deg)
    o_ref[0:N, :] = jnp.dot(x_ref[:] * dinv, w_ref[:],
                            preferred_element_type=jnp.float32)
    o_ref[N:NP, :] = jnp.zeros((NP - N, D), jnp.float32)


def _out_body(z_ref, xlin_ref, degt_ref, b_ref, o_ref):
    deg = degt_ref[:N, 0:1] + degt_ref[:N, 1:2] + 1.0
    dinv = lax.rsqrt(deg)
    zsum = z_ref[0, 0:N] + z_ref[1, 0:N] + xlin_ref[0:N, :]
    o_ref[:] = jnp.maximum(zsum * dinv + b_ref[:], 0.0)


def kernel(x, edge_index, W, b):
    ei = edge_index.astype(jnp.int32)
    nw = NC * NS
    src = jnp.concatenate([ei[0], jnp.zeros((EP - E,), jnp.int32)])
    dst = jnp.concatenate([ei[1], jnp.full((EP - E,), N, jnp.int32)])
    src3 = src.reshape(NCHUNK, nw, CHUNK).swapaxes(0, 1)
    dst3 = dst.reshape(NCHUNK, nw, CHUNK).swapaxes(0, 1)
    dst = dst3.reshape(EP)

    deg2 = _deg_kernel(dst3)
    degt = deg2.T

    xlin = pl.pallas_call(
        _lin_body,
        out_shape=jax.ShapeDtypeStruct((NP, D), jnp.float32),
    )(x, W, degt)

    z2 = _msg_kernel(src3, dst, xlin)

    out = pl.pallas_call(
        _out_body,
        out_shape=jax.ShapeDtypeStruct((N, D), jnp.float32),
    )(z2, xlin, degt, b.reshape(1, D))
    return out
